# baseline (speedup 1.0000x reference)
import functools

import jax
import jax.numpy as jnp
import numpy as np
from jax import lax
from jax.experimental import pallas as pl
from jax.experimental.pallas import tpu as pltpu
from jax.experimental.pallas import tpu_sc as plsc

N = 10000
E = 320000
D = 128
H = 128
R = 64
G = 64
EPS = 1e-5

NP_ = 10240
NB = 1280
NBLK = NP_ // NB
NWORK = 32
K = 128
NCHUNK = 80
EPW = NCHUNK * K
EPAD = EPW * NWORK
NPAIR = NCHUNK // 2
ROWS_PER_TILE = NP_ // 16

_mesh = plsc.VectorSubcoreMesh(core_axis_name="c", subcore_axis_name="s")


@functools.partial(
    pl.kernel,
    out_type=jax.ShapeDtypeStruct((NWORK, NP_), jnp.float32),
    mesh=_mesh,
    compiler_params=pltpu.CompilerParams(needs_layout_passes=False),
    scratch_types=[
        pltpu.VMEM((K,), jnp.int32),
        pltpu.VMEM((NP_,), jnp.float32),
    ],
)
def _deg_partials(dst_hbm, out_hbm, dst_v, deg_v):
    c = lax.axis_index("c")
    s = lax.axis_index("s")
    wid = s * 2 + c

    def zero(i, _):
        deg_v[pl.ds(i * 16, 16)] = jnp.zeros((16,), jnp.float32)
        return 0
    lax.fori_loop(0, NP_ // 16, zero, 0)

    ones16 = jnp.ones((16,), jnp.float32)

    def body(i, _):
        off = pl.multiple_of((i * NWORK + wid) * K, 8)
        pltpu.sync_copy(dst_hbm.at[pl.ds(off, K)], dst_v)
        for k in range(K // 16):
            idx = dst_v[pl.ds(k * 16, 16)]
            plsc.addupdate_scatter(deg_v, [idx], ones16)
        return 0
    lax.fori_loop(0, NCHUNK, body, 0)

    pltpu.sync_copy(deg_v, out_hbm.at[wid])


@functools.partial(
    pl.kernel,
    out_type=jax.ShapeDtypeStruct((2, NP_, D), jnp.float32),
    mesh=_mesh,
    scratch_types=[
        pltpu.VMEM((K,), jnp.int32),
        pltpu.VMEM((K,), jnp.int32),
        pltpu.VMEM((K,), jnp.int32),
        pltpu.VMEM((K,), jnp.int32),
        pltpu.VMEM((K, D), jnp.float32),
        pltpu.VMEM((K, D), jnp.float32),
        pltpu.VMEM_SHARED((NP_, D), jnp.float32),
        pltpu.SemaphoreType.DMA,
        pltpu.SemaphoreType.DMA,
        pltpu.SemaphoreType.DMA,
        pltpu.SemaphoreType.DMA,
    ],
)
def _edge_scatter(z_hbm, src_hbm, dst_hbm, out_hbm,
                  src_a, dst_a, src_b, dst_b, rows_a, rows_b, acc,
                  gsa, gsb, ssa, ssb):
    c = lax.axis_index("c")
    s = lax.axis_index("s")
    wid = s * 2 + c

    def zrow(j, _):
        def zcol(k, _):
            rows_a[j, pl.ds(k * 16, 16)] = jnp.zeros((16,), jnp.float32)
            return 0
        return lax.fori_loop(0, D // 16, zcol, 0)
    lax.fori_loop(0, K, zrow, 0)

    row0 = s * ROWS_PER_TILE
    for t in range(ROWS_PER_TILE // K):
        pltpu.sync_copy(rows_a, acc.at[pl.ds(row0 + t * K, K)])
    plsc.subcore_barrier()

    def ld(i, sv, dv):
        off = pl.multiple_of((i * NWORK + wid) * K, 8)
        pltpu.sync_copy(src_hbm.at[pl.ds(off, K)], sv)
        pltpu.sync_copy(dst_hbm.at[pl.ds(off, K)], dv)

    def body(j, _):
        @pl.when(j > 0)
        def _():
            pltpu.make_async_copy(rows_a, acc.at[dst_a], ssa).wait()
        ld(2 * j, src_a, dst_a)
        pltpu.async_copy(z_hbm.at[src_a], rows_a, gsa).wait()
        pltpu.async_copy(rows_a, acc.at[dst_a], ssa, add=True)

        @pl.when(j > 0)
        def _():
            pltpu.make_async_copy(rows_b, acc.at[dst_b], ssb).wait()
        ld(2 * j + 1, src_b, dst_b)
        pltpu.async_copy(z_hbm.at[src_b], rows_b, gsb).wait()
        pltpu.async_copy(rows_b, acc.at[dst_b], ssb, add=True)
        return 0
    lax.fori_loop(0, NPAIR, body, 0)
    pltpu.make_async_copy(rows_a, acc.at[dst_a], ssa).wait()
    pltpu.make_async_copy(rows_b, acc.at[dst_b], ssb).wait()

    plsc.subcore_barrier()
    pltpu.sync_copy(acc.at[pl.ds(row0, ROWS_PER_TILE)],
                    out_hbm.at[c, pl.ds(row0, ROWS_PER_TILE)])


def _z1_body(x_ref, w_ref, degp_ref, z_ref):
    deg = jnp.sum(degp_ref[...], axis=0) + 1.0
    dinv = lax.rsqrt(deg)
    ht = jnp.dot(x_ref[...], w_ref[...], preferred_element_type=jnp.float32)
    z_ref[...] = ht * dinv[:, None]


def _h1z2_body(p_ref, z1_ref, degp_ref, b1_ref, gamma_ref, beta_ref, w2_ref, z2_ref):
    deg = jnp.sum(degp_ref[...], axis=0) + 1.0
    dinv = lax.rsqrt(deg)
    agg = (p_ref[0] + p_ref[1] + z1_ref[...]) * dinv[:, None] + b1_ref[...]
    h = agg * (gamma_ref[...] * (1.0 / np.sqrt(1.0 + EPS))) + beta_ref[...]
    h = jnp.maximum(h, 0.0)
    ht2 = jnp.dot(h, w2_ref[...], preferred_element_type=jnp.float32)
    z2_ref[...] = ht2 * dinv[:, None]


def _final_body(p_ref, z2_ref, degp_ref, b2_ref, batch_ref, rst_ref,
                wg_ref, bg_ref, wr_ref, br_ref, wc_ref, bc_ref,
                out_ref, psum, pcnt):
    i = pl.program_id(0)

    @pl.when(i == 0)
    def _():
        psum[...] = jnp.zeros_like(psum)
        pcnt[...] = jnp.zeros_like(pcnt)

    deg = jnp.sum(degp_ref[...], axis=0) + 1.0
    dinv = lax.rsqrt(deg)
    h2 = (p_ref[0] + p_ref[1] + z2_ref[...]) * dinv[:, None] + b2_ref[...]
    b = batch_ref[0, 0]
    gids = lax.broadcasted_iota(jnp.int32, (G, NB), 0)
    mask = (gids == jnp.broadcast_to(b[None, :], (G, NB))).astype(jnp.float32)
    psum[...] += jnp.dot(mask, h2, preferred_element_type=jnp.float32)
    pcnt[...] += jnp.broadcast_to(jnp.sum(mask, axis=1)[:, None], (G, H))

    @pl.when(i == pl.num_programs(0) - 1)
    def _():
        pooled = psum[...] / jnp.maximum(pcnt[...], 1.0)
        xg = jnp.maximum(
            jnp.dot(pooled, wg_ref[...], preferred_element_type=jnp.float32)
            + bg_ref[...], 0.0)
        xr = jnp.maximum(
            jnp.dot(rst_ref[...], wr_ref[...], preferred_element_type=jnp.float32)
            + br_ref[...], 0.0)
        comb = jnp.concatenate([xg, xr], axis=1)
        out_ref[...] = (jnp.dot(comb, wc_ref[...],
                                preferred_element_type=jnp.float32)
                        + bc_ref[...])


def _row_spec():
    return pl.BlockSpec((NB, D), lambda i: (i, 0))


def _full_spec(shape):
    nd = len(shape)
    return pl.BlockSpec(shape, lambda i: (0,) * nd)


def kernel(x, edge_index, batch, rst, W1, b1, gamma, beta, W2, b2, Wg, bg, Wr, br, Wc, bc):
    f32 = jnp.float32
    x_pad = jnp.concatenate([x, jnp.zeros((NP_ - N, D), f32)], axis=0)
    fill = jnp.full((EPAD - E,), N, jnp.int32)
    src = jnp.concatenate([edge_index[0], fill])
    dst = jnp.concatenate([edge_index[1], fill])
    batch_pad = jnp.concatenate([batch, jnp.full((NP_ - N,), G, jnp.int32)])
    batch3d = batch_pad.reshape(NBLK, 1, NB)
    b1r, gr, br_ = b1.reshape(1, H), gamma.reshape(1, H), beta.reshape(1, H)
    b2r = b2.reshape(1, H)
    bgr, brr = bg.reshape(1, H // 2), br.reshape(1, H // 2)
    wc_pad = jnp.concatenate([Wc, jnp.zeros((H, 128 - 2), f32)], axis=1)
    bc_pad = jnp.concatenate([bc, jnp.zeros((128 - 2,), f32)]).reshape(1, 128)

    degp = _deg_partials(dst)

    z1 = pl.pallas_call(
        _z1_body,
        grid=(NBLK,),
        in_specs=[
            _row_spec(),
            _full_spec((D, H)),
            pl.BlockSpec((NWORK, NB), lambda i: (0, i)),
        ],
        out_specs=_row_spec(),
        out_shape=jax.ShapeDtypeStruct((NP_, H), f32),
    )(x_pad, W1, degp)

    p1 = _edge_scatter(z1, src, dst)

    z2 = pl.pallas_call(
        _h1z2_body,
        grid=(NBLK,),
        in_specs=[
            pl.BlockSpec((2, NB, H), lambda i: (0, i, 0)),
            _row_spec(),
            pl.BlockSpec((NWORK, NB), lambda i: (0, i)),
            _full_spec((1, H)),
            _full_spec((1, H)),
            _full_spec((1, H)),
            _full_spec((H, H)),
        ],
        out_specs=_row_spec(),
        out_shape=jax.ShapeDtypeStruct((NP_, H), f32),
    )(p1, z1, degp, b1r, gr, br_, W2)

    p2 = _edge_scatter(z2, src, dst)

    out_pad = pl.pallas_call(
        _final_body,
        grid=(NBLK,),
        in_specs=[
            pl.BlockSpec((2, NB, H), lambda i: (0, i, 0)),
            _row_spec(),
            pl.BlockSpec((NWORK, NB), lambda i: (0, i)),
            _full_spec((1, H)),
            pl.BlockSpec((1, 1, NB), lambda i: (i, 0, 0)),
            _full_spec((G, R)),
            _full_spec((H, H // 2)),
            _full_spec((1, H // 2)),
            _full_spec((R, H // 2)),
            _full_spec((1, H // 2)),
            _full_spec((H, 128)),
            _full_spec((1, 128)),
        ],
        out_specs=_full_spec((G, 128)),
        out_shape=jax.ShapeDtypeStruct((G, 128), f32),
        scratch_shapes=[
            pltpu.VMEM((G, H), f32),
            pltpu.VMEM((G, H), f32),
        ],
    )(p2, z2, degp, b2r, batch3d, rst, Wg, bgr, Wr, brr, wc_pad, bc_pad)

    return out_pad[:, :2]


# speedup vs baseline: 1.2680x; 1.2680x over previous
import functools

import jax
import jax.numpy as jnp
import numpy as np
from jax import lax
from jax.experimental import pallas as pl
from jax.experimental.pallas import tpu as pltpu
from jax.experimental.pallas import tpu_sc as plsc

N = 10000
E = 320000
D = 128
H = 128
R = 64
G = 64
EPS = 1e-5

NP_ = 10240
NB = 1280
NBLK = NP_ // NB
NWORK = 32
K = 128
EPW = ((E + NWORK * K - 1) // (NWORK * K)) * K
EPAD = EPW * NWORK
NCHUNK = EPW // K
ROWS_PER_TILE = NP_ // 16

_mesh = plsc.VectorSubcoreMesh(core_axis_name="c", subcore_axis_name="s")


@functools.partial(
    pl.kernel,
    out_type=jax.ShapeDtypeStruct((NWORK, NP_), jnp.float32),
    mesh=_mesh,
    compiler_params=pltpu.CompilerParams(needs_layout_passes=False),
    scratch_types=[
        pltpu.VMEM((K,), jnp.int32),
        pltpu.VMEM((NP_,), jnp.float32),
    ],
)
def _deg_partials(dst_hbm, out_hbm, dst_v, deg_v):
    c = lax.axis_index("c")
    s = lax.axis_index("s")
    wid = s * 2 + c

    def zero(i, _):
        deg_v[pl.ds(i * 16, 16)] = jnp.zeros((16,), jnp.float32)
        return 0
    lax.fori_loop(0, NP_ // 16, zero, 0)

    ones16 = jnp.ones((16,), jnp.float32)
    base = wid * EPW

    def body(i, _):
        off = pl.multiple_of(base + i * K, 8)
        pltpu.sync_copy(dst_hbm.at[pl.ds(off, K)], dst_v)
        for k in range(K // 16):
            idx = dst_v[pl.ds(k * 16, 16)]
            plsc.addupdate_scatter(deg_v, [idx], ones16)
        return 0
    lax.fori_loop(0, NCHUNK, body, 0)

    pltpu.sync_copy(deg_v, out_hbm.at[wid])


@functools.partial(
    pl.kernel,
    out_type=jax.ShapeDtypeStruct((2, NP_, D), jnp.float32),
    mesh=_mesh,
    scratch_types=[
        pltpu.VMEM((K,), jnp.int32),
        pltpu.VMEM((K,), jnp.int32),
        pltpu.VMEM((K, D), jnp.float32),
        pltpu.VMEM_SHARED((NP_, D), jnp.float32),
        pltpu.SemaphoreType.DMA,
    ],
)
def _edge_scatter(z_hbm, src_hbm, dst_hbm, out_hbm, src_v, dst_v, rows_v, acc, sem):
    c = lax.axis_index("c")
    s = lax.axis_index("s")
    wid = s * 2 + c

    def zrow(j, _):
        def zcol(k, _):
            rows_v[j, pl.ds(k * 16, 16)] = jnp.zeros((16,), jnp.float32)
            return 0
        return lax.fori_loop(0, D // 16, zcol, 0)
    lax.fori_loop(0, K, zrow, 0)

    row0 = s * ROWS_PER_TILE
    for t in range(ROWS_PER_TILE // K):
        pltpu.sync_copy(rows_v, acc.at[pl.ds(row0 + t * K, K)])
    plsc.subcore_barrier()

    base = wid * EPW

    def body(i, _):
        off = pl.multiple_of(base + i * K, 8)
        pltpu.sync_copy(src_hbm.at[pl.ds(off, K)], src_v)
        pltpu.sync_copy(dst_hbm.at[pl.ds(off, K)], dst_v)
        pltpu.async_copy(z_hbm.at[src_v], rows_v, sem).wait()
        pltpu.sync_copy(rows_v, acc.at[dst_v], add=True)
        return 0
    lax.fori_loop(0, NCHUNK, body, 0)

    plsc.subcore_barrier()
    pltpu.sync_copy(acc.at[pl.ds(row0, ROWS_PER_TILE)],
                    out_hbm.at[c, pl.ds(row0, ROWS_PER_TILE)])


def _z1_body(x_ref, w_ref, degp_ref, z_ref):
    deg = jnp.sum(degp_ref[...], axis=0) + 1.0
    dinv = lax.rsqrt(deg)
    ht = jnp.dot(x_ref[...], w_ref[...], preferred_element_type=jnp.float32)
    z_ref[...] = ht * dinv[:, None]


def _h1z2_body(p_ref, z1_ref, degp_ref, b1_ref, gamma_ref, beta_ref, w2_ref, z2_ref):
    deg = jnp.sum(degp_ref[...], axis=0) + 1.0
    dinv = lax.rsqrt(deg)
    agg = (p_ref[0] + p_ref[1] + z1_ref[...]) * dinv[:, None] + b1_ref[...]
    h = agg * (gamma_ref[...] * (1.0 / np.sqrt(1.0 + EPS))) + beta_ref[...]
    h = jnp.maximum(h, 0.0)
    ht2 = jnp.dot(h, w2_ref[...], preferred_element_type=jnp.float32)
    z2_ref[...] = ht2 * dinv[:, None]


def _final_body(p_ref, z2_ref, degp_ref, b2_ref, batch_ref, rst_ref,
                wg_ref, bg_ref, wr_ref, br_ref, wc_ref, bc_ref,
                out_ref, psum, pcnt):
    i = pl.program_id(0)

    @pl.when(i == 0)
    def _():
        psum[...] = jnp.zeros_like(psum)
        pcnt[...] = jnp.zeros_like(pcnt)

    deg = jnp.sum(degp_ref[...], axis=0) + 1.0
    dinv = lax.rsqrt(deg)
    h2 = (p_ref[0] + p_ref[1] + z2_ref[...]) * dinv[:, None] + b2_ref[...]
    b = batch_ref[0, 0]
    gids = lax.broadcasted_iota(jnp.int32, (G, NB), 0)
    mask = (gids == jnp.broadcast_to(b[None, :], (G, NB))).astype(jnp.float32)
    psum[...] += jnp.dot(mask, h2, preferred_element_type=jnp.float32)
    pcnt[...] += jnp.broadcast_to(jnp.sum(mask, axis=1)[:, None], (G, H))

    @pl.when(i == pl.num_programs(0) - 1)
    def _():
        pooled = psum[...] / jnp.maximum(pcnt[...], 1.0)
        xg = jnp.maximum(
            jnp.dot(pooled, wg_ref[...], preferred_element_type=jnp.float32)
            + bg_ref[...], 0.0)
        xr = jnp.maximum(
            jnp.dot(rst_ref[...], wr_ref[...], preferred_element_type=jnp.float32)
            + br_ref[...], 0.0)
        comb = jnp.concatenate([xg, xr], axis=1)
        out_ref[...] = (jnp.dot(comb, wc_ref[...],
                                preferred_element_type=jnp.float32)
                        + bc_ref[...])


def _row_spec():
    return pl.BlockSpec((NB, D), lambda i: (i, 0))


def _full_spec(shape):
    nd = len(shape)
    return pl.BlockSpec(shape, lambda i: (0,) * nd)


def kernel(x, edge_index, batch, rst, W1, b1, gamma, beta, W2, b2, Wg, bg, Wr, br, Wc, bc):
    f32 = jnp.float32
    x_pad = jnp.concatenate([x, jnp.zeros((NP_ - N, D), f32)], axis=0)
    fill = jnp.full((EPAD - E,), N, jnp.int32)
    src = jnp.concatenate([edge_index[0], fill])
    dst = jnp.concatenate([edge_index[1], fill])
    batch_pad = jnp.concatenate([batch, jnp.full((NP_ - N,), G, jnp.int32)])
    batch3d = batch_pad.reshape(NBLK, 1, NB)
    b1r, gr, br_ = b1.reshape(1, H), gamma.reshape(1, H), beta.reshape(1, H)
    b2r = b2.reshape(1, H)
    bgr, brr = bg.reshape(1, H // 2), br.reshape(1, H // 2)
    wc_pad = jnp.concatenate([Wc, jnp.zeros((H, 128 - 2), f32)], axis=1)
    bc_pad = jnp.concatenate([bc, jnp.zeros((128 - 2,), f32)]).reshape(1, 128)

    degp = _deg_partials(dst)

    z1 = pl.pallas_call(
        _z1_body,
        grid=(NBLK,),
        in_specs=[
            _row_spec(),
            _full_spec((D, H)),
            pl.BlockSpec((NWORK, NB), lambda i: (0, i)),
        ],
        out_specs=_row_spec(),
        out_shape=jax.ShapeDtypeStruct((NP_, H), f32),
    )(x_pad, W1, degp)

    p1 = _edge_scatter(z1, src, dst)

    z2 = pl.pallas_call(
        _h1z2_body,
        grid=(NBLK,),
        in_specs=[
            pl.BlockSpec((2, NB, H), lambda i: (0, i, 0)),
            _row_spec(),
            pl.BlockSpec((NWORK, NB), lambda i: (0, i)),
            _full_spec((1, H)),
            _full_spec((1, H)),
            _full_spec((1, H)),
            _full_spec((H, H)),
        ],
        out_specs=_row_spec(),
        out_shape=jax.ShapeDtypeStruct((NP_, H), f32),
    )(p1, z1, degp, b1r, gr, br_, W2)

    p2 = _edge_scatter(z2, src, dst)

    out_pad = pl.pallas_call(
        _final_body,
        grid=(NBLK,),
        in_specs=[
            pl.BlockSpec((2, NB, H), lambda i: (0, i, 0)),
            _row_spec(),
            pl.BlockSpec((NWORK, NB), lambda i: (0, i)),
            _full_spec((1, H)),
            pl.BlockSpec((1, 1, NB), lambda i: (i, 0, 0)),
            _full_spec((G, R)),
            _full_spec((H, H // 2)),
            _full_spec((1, H // 2)),
            _full_spec((R, H // 2)),
            _full_spec((1, H // 2)),
            _full_spec((H, 128)),
            _full_spec((1, 128)),
        ],
        out_specs=_full_spec((G, 128)),
        out_shape=jax.ShapeDtypeStruct((G, 128), f32),
        scratch_shapes=[
            pltpu.VMEM((G, H), f32),
            pltpu.VMEM((G, H), f32),
        ],
    )(p2, z2, degp, b2r, batch3d, rst, Wg, bgr, Wr, brr, wc_pad, bc_pad)

    return out_pad[:, :2]
